# baseline (speedup 1.0000x reference)
import functools

import jax
import jax.numpy as jnp
from jax import lax
from jax.experimental import pallas as pl
from jax.experimental.pallas import tpu as pltpu
from jax.experimental.pallas import tpu_sc as plsc

_N = 10000
_E = 160000
_H = 2
_C = 128

_NP = 10240
_ET = _E // 16
_RK = 80
_NCH = _ET // _RK


def _lrelu01(t):
    return jnp.where(t > 0, t, 0.01 * t)


def _k1_body(x_ref, w_ref, aall_ref, xp0_ref, xp1_ref, a_ref):
    xp = jnp.dot(x_ref[...], w_ref[...], preferred_element_type=jnp.float32)
    xp0_ref[...] = xp[:, :_C]
    xp1_ref[...] = xp[:, _C:]
    a_ref[...] = jnp.dot(xp, aall_ref[...], preferred_element_type=jnp.float32)


def _k1(x, W, aall):
    bm = 2000
    grid = (_N // bm,)
    return pl.pallas_call(
        _k1_body,
        grid=grid,
        in_specs=[
            pl.BlockSpec((bm, 512), lambda i: (i, 0)),
            pl.BlockSpec((512, _H * _C), lambda i: (0, 0)),
            pl.BlockSpec((_H * _C, 8), lambda i: (0, 0)),
        ],
        out_specs=[
            pl.BlockSpec((bm, _C), lambda i: (i, 0)),
            pl.BlockSpec((bm, _C), lambda i: (i, 0)),
            pl.BlockSpec((bm, 8), lambda i: (i, 0)),
        ],
        out_shape=[
            jax.ShapeDtypeStruct((_N, _C), jnp.float32),
            jax.ShapeDtypeStruct((_N, _C), jnp.float32),
            jax.ShapeDtypeStruct((_N, 8), jnp.float32),
        ],
    )(x, W, aall)


_GRP = 400
_NGRP = _ET // _GRP


def _sc_edge_body(xpf, asrc2, adst2, src_h, dst3, den02, zblk, out,
                  idx_sb, dst2b, as_t, ad_t, den_t, rows, coef_t, idxrow2,
                  den_sh, accum, sem):
    c = lax.axis_index("c")
    s = lax.axis_index("s")
    shift = c * _NP

    pltpu.sync_copy(asrc2.at[c], as_t)
    pltpu.sync_copy(adst2.at[c], ad_t)
    pltpu.sync_copy(den02, den_t)
    pltpu.sync_copy(zblk, accum.at[pl.ds(s * 640, 640)])

    @pl.when(s == 0)
    def _():
        pltpu.sync_copy(den02, den_sh)

    def _rowidx_body(j, _):
        idxrow2[j, pl.ds(0, 16)] = lax.iota(jnp.int32, 16) + j * 16
        return 0

    lax.fori_loop(0, 40, _rowidx_body, 0)

    def _den_grp(g, _):
        ebase = s * _ET + g * _GRP
        pltpu.sync_copy(src_h.at[pl.ds(ebase, _GRP)], idx_sb)
        pltpu.sync_copy(dst3.at[s, pl.ds(g * (_GRP // _RK), _GRP // _RK)],
                        dst2b)

        def _den_body(i, _):
            vs = idx_sb[pl.ds(i * 16, 16)]
            vd = dst2b[i // 5, pl.ds((i % 5) * 16, 16)]
            al = plsc.load_gather(as_t, [vs]) + plsc.load_gather(ad_t, [vd])
            al = jnp.where(al > 0, al, 0.2 * al)
            ex = jnp.exp(al)
            plsc.addupdate_scatter(
                den_t, [lax.shift_right_logical(vd, 4), vd & 15], ex)
            return 0

        lax.fori_loop(0, _GRP // 16, _den_body, 0)
        return 0

    lax.fori_loop(0, _NGRP, _den_grp, 0)
    plsc.subcore_barrier()

    def _red_body(j, _):
        pltpu.sync_copy(den_t.at[pl.ds(j * 16, 16)],
                        den_sh.at[idxrow2.at[j]], add=True)
        return 0

    lax.fori_loop(0, 40, _red_body, 0)
    plsc.subcore_barrier()
    pltpu.sync_copy(den_sh, den_t)

    def _agg_grp(g, _):
        ebase = s * _ET + g * _GRP
        pltpu.sync_copy(src_h.at[pl.ds(ebase, _GRP)], idx_sb)
        pltpu.sync_copy(dst3.at[s, pl.ds(g * (_GRP // _RK), _GRP // _RK)],
                        dst2b)

        def _shift_body(i, _):
            idx_sb[pl.ds(i * 16, 16)] = idx_sb[pl.ds(i * 16, 16)] + shift
            return 0

        lax.fori_loop(0, _GRP // 16, _shift_body, 0)

        def _chunk_body(k, _):
            pltpu.async_copy(xpf.at[idx_sb.at[pl.ds(k * _RK, _RK)]],
                             rows, sem).wait()

            @plsc.parallel_loop(0, _RK // 16, 1, unroll=1)
            def _coef_body(q):
                i = k * (_RK // 16) + q
                vs = idx_sb[pl.ds(i * 16, 16)] - shift
                vd = dst2b[i // 5, pl.ds((i % 5) * 16, 16)]
                al = plsc.load_gather(as_t, [vs]) + plsc.load_gather(ad_t, [vd])
                al = jnp.where(al > 0, al, 0.2 * al)
                ex = jnp.exp(al)
                den = plsc.load_gather(
                    den_t, [lax.shift_right_logical(vd, 4), vd & 15])
                coef_t[pl.ds(q * 16, 16)] = ex / (den + 1e-16)

            @plsc.parallel_loop(0, _RK, 1, unroll=8)
            def _row_body(r):
                cs = plsc.load_gather(coef_t, [jnp.full((16,), r, jnp.int32)])
                for k2 in range(_C // 16):
                    rows[r, pl.ds(k2 * 16, 16)] = \
                        rows[r, pl.ds(k2 * 16, 16)] * cs

            pltpu.sync_copy(rows, accum.at[dst2b.at[k]], add=True)
            return 0

        lax.fori_loop(0, _GRP // _RK, _chunk_body, 0)
        return 0

    lax.fori_loop(0, _NGRP, _agg_grp, 0)

    plsc.subcore_barrier()
    pltpu.sync_copy(accum.at[pl.ds(s * 640, 640)],
                    out.at[c, pl.ds(s * 640, 640)])


def _sc_edge(xpf, asrc2, adst2, src, dst3, den02, zblk):
    mesh = plsc.VectorSubcoreMesh(core_axis_name="c", subcore_axis_name="s")
    f = functools.partial(
        pl.kernel,
        mesh=mesh,
        compiler_params=pltpu.CompilerParams(
            use_tc_tiling_on_sc=False, needs_layout_passes=False),
        out_type=jax.ShapeDtypeStruct((_H, _NP, _C), jnp.float32),
        scratch_types=[
            pltpu.VMEM((_GRP,), jnp.int32),
            pltpu.VMEM((_GRP // _RK, _RK), jnp.int32),
            pltpu.VMEM((_NP,), jnp.float32),
            pltpu.VMEM((_NP,), jnp.float32),
            pltpu.VMEM((_NP // 16, 16), jnp.float32),
            pltpu.VMEM((_RK, _C), jnp.float32),
            pltpu.VMEM((_RK,), jnp.float32),
            pltpu.VMEM((40, 16), jnp.int32),
            pltpu.VMEM_SHARED((_NP // 16, 16), jnp.float32),
            pltpu.VMEM_SHARED((_NP, _C), jnp.float32),
            pltpu.SemaphoreType.DMA,
        ],
    )(_sc_edge_body)
    return f(xpf, asrc2, adst2, src, dst3, den02, zblk)


def _k3_body(o0_ref, o1_ref, bc0_ref, bc1_ref, wa0_ref, wa1_ref, ba_ref,
             w1_ref, b1_ref, w2_ref, b2_ref, w3_ref, b3_ref, y_ref):
    t0 = _lrelu01(o0_ref[...] + bc0_ref[...])
    t1 = _lrelu01(o1_ref[...] + bc1_ref[...])
    h = jnp.dot(t0, wa0_ref[...], preferred_element_type=jnp.float32)
    h = h + jnp.dot(t1, wa1_ref[...], preferred_element_type=jnp.float32)
    h = _lrelu01(h + ba_ref[...])
    h = _lrelu01(jnp.dot(h, w1_ref[...], preferred_element_type=jnp.float32)
                 + b1_ref[...])
    h = _lrelu01(jnp.dot(h, w2_ref[...], preferred_element_type=jnp.float32)
                 + b2_ref[...])
    y_ref[...] = jnp.dot(h, w3_ref[...], preferred_element_type=jnp.float32) \
        + b3_ref[...]


def _k3(o0, o1, bc0, bc1, Wa0, Wa1, ba, W1, b1, W2, b2, W3p, b3p):
    bm = 2000
    grid = (_N // bm,)
    full = lambda r, c: pl.BlockSpec((r, c), lambda i: (0, 0))
    return pl.pallas_call(
        _k3_body,
        grid=grid,
        in_specs=[
            pl.BlockSpec((bm, _C), lambda i: (i, 0)),
            pl.BlockSpec((bm, _C), lambda i: (i, 0)),
            full(1, _C), full(1, _C),
            full(_C, _C), full(_C, _C), full(1, _C),
            full(_C, 64), full(1, 64),
            full(64, 32), full(1, 32),
            full(32, 8), full(1, 8),
        ],
        out_specs=pl.BlockSpec((bm, 8), lambda i: (i, 0)),
        out_shape=jax.ShapeDtypeStruct((_N, 8), jnp.float32),
    )(o0, o1, bc0, bc1, Wa0, Wa1, ba, W1, b1, W2, b2, W3p, b3p)


def _k4_body(y_ref, yt_ref, out_ref):
    yi = y_ref[...]
    yjt = yt_ref[...]
    g = jnp.dot(yi, yjt, preferred_element_type=jnp.float32)
    ri = jnp.sum(yi * yi, axis=1, keepdims=True)
    rj = jnp.sum(yjt * yjt, axis=0, keepdims=True)
    d2 = jnp.maximum(ri + rj - 2.0 * g, 0.0)
    pos = d2 > 0
    out_ref[...] = jnp.where(pos, jnp.sqrt(jnp.where(pos, d2, 1.0)), 0.0)


def _k4(y, yT):
    bm = 200
    grid = (_N // bm,)
    return pl.pallas_call(
        _k4_body,
        grid=grid,
        in_specs=[
            pl.BlockSpec((bm, 8), lambda i: (i, 0)),
            pl.BlockSpec((8, _N), lambda i: (0, 0)),
        ],
        out_specs=pl.BlockSpec((bm, _N), lambda i: (i, 0)),
        out_shape=jax.ShapeDtypeStruct((_N, _N), jnp.float32),
    )(y, yT)


def kernel(x, edge_index, W, att_src, att_dst, b_conv, Wa, ba, W1, b1,
           W2, b2, W3, b3):
    src = edge_index[0].astype(jnp.int32)
    dst = edge_index[1].astype(jnp.int32)

    av_src = att_src.reshape(_H, _C)
    av_dst = att_dst.reshape(_H, _C)
    aall = jnp.zeros((_H * _C, 8), jnp.float32)
    aall = aall.at[:_C, 0].set(av_src[0])
    aall = aall.at[_C:, 1].set(av_src[1])
    aall = aall.at[:_C, 2].set(av_dst[0])
    aall = aall.at[_C:, 3].set(av_dst[1])

    xp0, xp1, a = _k1(x, W, aall)

    pad = ((0, _NP - _N), (0, 0))
    xpf = jnp.concatenate([jnp.pad(xp0, pad), jnp.pad(xp1, pad)], axis=0)
    asrc2 = jnp.stack([jnp.pad(a[:, 0], (0, _NP - _N)),
                       jnp.pad(a[:, 1], (0, _NP - _N))])
    adst2 = jnp.stack([jnp.pad(a[:, 2], (0, _NP - _N)),
                       jnp.pad(a[:, 3], (0, _NP - _N))])
    dst3 = dst.reshape(16, _NCH, _RK)
    den02 = jnp.zeros((_NP // 16, 16), jnp.float32)
    zblk = jnp.zeros((640, _C), jnp.float32)

    o = _sc_edge(xpf, asrc2, adst2, src, dst3, den02, zblk)
    o0 = o[0, :_N]
    o1 = o[1, :_N]

    bc0 = b_conv[:_C].reshape(1, _C)
    bc1 = b_conv[_C:].reshape(1, _C)
    Wa0 = Wa[:_C]
    Wa1 = Wa[_C:]
    W3p = jnp.zeros((32, 8), jnp.float32).at[:, :3].set(W3)
    b3p = jnp.zeros((1, 8), jnp.float32).at[0, :3].set(b3)

    y = _k3(o0, o1, bc0, bc1, Wa0, Wa1, ba.reshape(1, -1),
            W1, b1.reshape(1, -1), W2, b2.reshape(1, -1), W3p, b3p)
    yT = y.T
    return _k4(y, yT)


# speedup vs baseline: 1.0335x; 1.0335x over previous
import functools

import jax
import jax.numpy as jnp
from jax import lax
from jax.experimental import pallas as pl
from jax.experimental.pallas import tpu as pltpu
from jax.experimental.pallas import tpu_sc as plsc

_N = 10000
_E = 160000
_H = 2
_C = 128

_NP = 10240
_ET = _E // 16
_RK = 80
_NCH = _ET // _RK


def _lrelu01(t):
    return jnp.where(t > 0, t, 0.01 * t)


def _k1_body(x_ref, w_ref, aall_ref, xp0_ref, xp1_ref, a_ref):
    xp = jnp.dot(x_ref[...], w_ref[...], preferred_element_type=jnp.float32)
    xp0_ref[...] = xp[:, :_C]
    xp1_ref[...] = xp[:, _C:]
    a_ref[...] = jnp.dot(xp, aall_ref[...], preferred_element_type=jnp.float32)


def _k1(x, W, aall):
    bm = 2000
    grid = (_N // bm,)
    return pl.pallas_call(
        _k1_body,
        grid=grid,
        in_specs=[
            pl.BlockSpec((bm, 512), lambda i: (i, 0)),
            pl.BlockSpec((512, _H * _C), lambda i: (0, 0)),
            pl.BlockSpec((_H * _C, 8), lambda i: (0, 0)),
        ],
        out_specs=[
            pl.BlockSpec((bm, _C), lambda i: (i, 0)),
            pl.BlockSpec((bm, _C), lambda i: (i, 0)),
            pl.BlockSpec((bm, 8), lambda i: (i, 0)),
        ],
        out_shape=[
            jax.ShapeDtypeStruct((_N, _C), jnp.float32),
            jax.ShapeDtypeStruct((_N, _C), jnp.float32),
            jax.ShapeDtypeStruct((_N, 8), jnp.float32),
        ],
    )(x, W, aall)


_GRP = 400
_NGRP = _ET // _GRP


def _sc_edge_body(xpf, asrc2, adst2, src_h, dst3, den02, zblk, out,
                  idx_sb, dst2b, as_t, ad_t, den_t, rows, coef_t, idxrow2,
                  den_sh, accum, sem):
    c = lax.axis_index("c")
    s = lax.axis_index("s")
    shift = c * _NP

    pltpu.sync_copy(asrc2.at[c], as_t)
    pltpu.sync_copy(adst2.at[c], ad_t)
    pltpu.sync_copy(den02, den_t)
    pltpu.sync_copy(zblk, accum.at[pl.ds(s * 640, 640)])

    @pl.when(s == 0)
    def _():
        pltpu.sync_copy(den02, den_sh)

    def _rowidx_body(j, _):
        idxrow2[j, pl.ds(0, 16)] = lax.iota(jnp.int32, 16) + j * 16
        return 0

    lax.fori_loop(0, 40, _rowidx_body, 0)

    def _den_grp(g, _):
        ebase = s * _ET + g * _GRP
        pltpu.sync_copy(src_h.at[pl.ds(ebase, _GRP)], idx_sb)
        pltpu.sync_copy(dst3.at[s, pl.ds(g * (_GRP // _RK), _GRP // _RK)],
                        dst2b)

        def _den_body(i, _):
            vs = idx_sb[pl.ds(i * 16, 16)]
            vd = dst2b[i // 5, pl.ds((i % 5) * 16, 16)]
            al = plsc.load_gather(as_t, [vs]) + plsc.load_gather(ad_t, [vd])
            al = jnp.where(al > 0, al, 0.2 * al)
            ex = jnp.exp(al)
            plsc.addupdate_scatter(
                den_t, [lax.shift_right_logical(vd, 4), vd & 15], ex)
            return 0

        lax.fori_loop(0, _GRP // 16, _den_body, 0)
        return 0

    lax.fori_loop(0, _NGRP, _den_grp, 0)
    plsc.subcore_barrier()

    def _red_body(j, _):
        pltpu.sync_copy(den_t.at[pl.ds(j * 16, 16)],
                        den_sh.at[idxrow2.at[j]], add=True)
        return 0

    lax.fori_loop(0, 40, _red_body, 0)
    plsc.subcore_barrier()
    pltpu.sync_copy(den_sh, den_t)

    def _agg_grp(g, _):
        ebase = s * _ET + g * _GRP
        pltpu.sync_copy(src_h.at[pl.ds(ebase, _GRP)], idx_sb)
        pltpu.sync_copy(dst3.at[s, pl.ds(g * (_GRP // _RK), _GRP // _RK)],
                        dst2b)

        def _shift_body(i, _):
            idx_sb[pl.ds(i * 16, 16)] = idx_sb[pl.ds(i * 16, 16)] + shift
            return 0

        lax.fori_loop(0, _GRP // 16, _shift_body, 0)

        def _chunk_body(k, _):
            pltpu.async_copy(xpf.at[idx_sb.at[pl.ds(k * _RK, _RK)]],
                             rows, sem).wait()

            @plsc.parallel_loop(0, _RK // 16, 1, unroll=1)
            def _coef_body(q):
                i = k * (_RK // 16) + q
                vs = idx_sb[pl.ds(i * 16, 16)] - shift
                vd = dst2b[i // 5, pl.ds((i % 5) * 16, 16)]
                al = plsc.load_gather(as_t, [vs]) + plsc.load_gather(ad_t, [vd])
                al = jnp.where(al > 0, al, 0.2 * al)
                ex = jnp.exp(al)
                den = plsc.load_gather(
                    den_t, [lax.shift_right_logical(vd, 4), vd & 15])
                coef_t[pl.ds(q * 16, 16)] = ex / (den + 1e-16)

            @plsc.parallel_loop(0, _RK, 1, unroll=8)
            def _row_body(r):
                cs = plsc.load_gather(coef_t, [jnp.full((16,), r, jnp.int32)])
                for k2 in range(_C // 16):
                    rows[r, pl.ds(k2 * 16, 16)] = \
                        rows[r, pl.ds(k2 * 16, 16)] * cs

            pltpu.sync_copy(rows, accum.at[dst2b.at[k]], add=True)
            return 0

        lax.fori_loop(0, _GRP // _RK, _chunk_body, 0)
        return 0

    lax.fori_loop(0, _NGRP, _agg_grp, 0)

    plsc.subcore_barrier()
    pltpu.sync_copy(accum.at[pl.ds(s * 640, 640)],
                    out.at[c, pl.ds(s * 640, 640)])


def _sc_edge(xpf, asrc2, adst2, src, dst3, den02, zblk):
    mesh = plsc.VectorSubcoreMesh(core_axis_name="c", subcore_axis_name="s")
    f = functools.partial(
        pl.kernel,
        mesh=mesh,
        compiler_params=pltpu.CompilerParams(
            use_tc_tiling_on_sc=False, needs_layout_passes=False),
        out_type=jax.ShapeDtypeStruct((_H, _NP, _C), jnp.float32),
        scratch_types=[
            pltpu.VMEM((_GRP,), jnp.int32),
            pltpu.VMEM((_GRP // _RK, _RK), jnp.int32),
            pltpu.VMEM((_NP,), jnp.float32),
            pltpu.VMEM((_NP,), jnp.float32),
            pltpu.VMEM((_NP // 16, 16), jnp.float32),
            pltpu.VMEM((_RK, _C), jnp.float32),
            pltpu.VMEM((_RK,), jnp.float32),
            pltpu.VMEM((40, 16), jnp.int32),
            pltpu.VMEM_SHARED((_NP // 16, 16), jnp.float32),
            pltpu.VMEM_SHARED((_NP, _C), jnp.float32),
            pltpu.SemaphoreType.DMA,
        ],
    )(_sc_edge_body)
    return f(xpf, asrc2, adst2, src, dst3, den02, zblk)


def _k3_body(o0_ref, o1_ref, bc0_ref, bc1_ref, wa0_ref, wa1_ref, ba_ref,
             w1_ref, b1_ref, w2_ref, b2_ref, w3_ref, b3_ref, y_ref):
    t0 = _lrelu01(o0_ref[...] + bc0_ref[...])
    t1 = _lrelu01(o1_ref[...] + bc1_ref[...])
    h = jnp.dot(t0, wa0_ref[...], preferred_element_type=jnp.float32)
    h = h + jnp.dot(t1, wa1_ref[...], preferred_element_type=jnp.float32)
    h = _lrelu01(h + ba_ref[...])
    h = _lrelu01(jnp.dot(h, w1_ref[...], preferred_element_type=jnp.float32)
                 + b1_ref[...])
    h = _lrelu01(jnp.dot(h, w2_ref[...], preferred_element_type=jnp.float32)
                 + b2_ref[...])
    y_ref[...] = jnp.dot(h, w3_ref[...], preferred_element_type=jnp.float32) \
        + b3_ref[...]


def _k3(o0, o1, bc0, bc1, Wa0, Wa1, ba, W1, b1, W2, b2, W3p, b3p):
    bm = 2000
    grid = (_N // bm,)
    full = lambda r, c: pl.BlockSpec((r, c), lambda i: (0, 0))
    return pl.pallas_call(
        _k3_body,
        grid=grid,
        in_specs=[
            pl.BlockSpec((bm, _C), lambda i: (i, 0)),
            pl.BlockSpec((bm, _C), lambda i: (i, 0)),
            full(1, _C), full(1, _C),
            full(_C, _C), full(_C, _C), full(1, _C),
            full(_C, 64), full(1, 64),
            full(64, 32), full(1, 32),
            full(32, 8), full(1, 8),
        ],
        out_specs=pl.BlockSpec((bm, 8), lambda i: (i, 0)),
        out_shape=jax.ShapeDtypeStruct((_N, 8), jnp.float32),
    )(o0, o1, bc0, bc1, Wa0, Wa1, ba, W1, b1, W2, b2, W3p, b3p)


def _k4_body(y_ref, yt_ref, out_ref):
    yi = y_ref[...]
    yjt = yt_ref[...]
    g = jnp.dot(yi, yjt, preferred_element_type=jnp.float32)
    ri = jnp.sum(yi * yi, axis=1, keepdims=True)
    rj = jnp.sum(yjt * yjt, axis=0, keepdims=True)
    d2 = jnp.maximum(ri + rj - 2.0 * g, 0.0)
    pos = d2 > 0
    out_ref[...] = jnp.where(pos, jnp.sqrt(jnp.where(pos, d2, 1.0)), 0.0)


def _k4(y, yT):
    bm = 400
    grid = (_N // bm,)
    return pl.pallas_call(
        _k4_body,
        grid=grid,
        in_specs=[
            pl.BlockSpec((bm, 8), lambda i: (i, 0)),
            pl.BlockSpec((8, _N), lambda i: (0, 0)),
        ],
        out_specs=pl.BlockSpec((bm, _N), lambda i: (i, 0)),
        out_shape=jax.ShapeDtypeStruct((_N, _N), jnp.float32),
    )(y, yT)


def kernel(x, edge_index, W, att_src, att_dst, b_conv, Wa, ba, W1, b1,
           W2, b2, W3, b3):
    src = edge_index[0].astype(jnp.int32)
    dst = edge_index[1].astype(jnp.int32)

    av_src = att_src.reshape(_H, _C)
    av_dst = att_dst.reshape(_H, _C)
    aall = jnp.zeros((_H * _C, 8), jnp.float32)
    aall = aall.at[:_C, 0].set(av_src[0])
    aall = aall.at[_C:, 1].set(av_src[1])
    aall = aall.at[:_C, 2].set(av_dst[0])
    aall = aall.at[_C:, 3].set(av_dst[1])

    xp0, xp1, a = _k1(x, W, aall)

    pad = ((0, _NP - _N), (0, 0))
    xpf = jnp.concatenate([jnp.pad(xp0, pad), jnp.pad(xp1, pad)], axis=0)
    asrc2 = jnp.stack([jnp.pad(a[:, 0], (0, _NP - _N)),
                       jnp.pad(a[:, 1], (0, _NP - _N))])
    adst2 = jnp.stack([jnp.pad(a[:, 2], (0, _NP - _N)),
                       jnp.pad(a[:, 3], (0, _NP - _N))])
    dst3 = dst.reshape(16, _NCH, _RK)
    den02 = jnp.zeros((_NP // 16, 16), jnp.float32)
    zblk = jnp.zeros((640, _C), jnp.float32)

    o = _sc_edge(xpf, asrc2, adst2, src, dst3, den02, zblk)
    o0 = o[0, :_N]
    o1 = o[1, :_N]

    bc0 = b_conv[:_C].reshape(1, _C)
    bc1 = b_conv[_C:].reshape(1, _C)
    Wa0 = Wa[:_C]
    Wa1 = Wa[_C:]
    W3p = jnp.zeros((32, 8), jnp.float32).at[:, :3].set(W3)
    b3p = jnp.zeros((1, 8), jnp.float32).at[0, :3].set(b3)

    y = _k3(o0, o1, bc0, bc1, Wa0, Wa1, ba.reshape(1, -1),
            W1, b1.reshape(1, -1), W2, b2.reshape(1, -1), W3p, b3p)
    yT = y.T
    return _k4(y, yT)
